# Initial kernel scaffold; baseline (speedup 1.0000x reference)
import jax
import jax.numpy as jnp
from jax import lax
from jax.experimental import pallas as pl
from jax.experimental.pallas import tpu as pltpu
from jax.experimental.pallas import tpu_sc as plsc

B = 16384
D_DRV, D_WEEK, D_TIME = 16, 3, 8
D_OUT = D_DRV + D_WEEK + D_TIME + 1
NW = 32
BPW = B // NW
NCHUNK = BPW // 128

DIST_MEAN = 10.0
DIST_STD = 5.0


def _body(drv_idx_hbm, wk_idx_hbm, tm_idx_hbm, dist_hbm,
          drv_tab_hbm, wk_tab_hbm, tm_tab_hbm, out_hbm,
          didx_v, tidx_v, widx_v, dist_v, d_rows, t_rows, wk_tab_v, out_v,
          sem_d, sem_t):
    wid = lax.axis_index("s") * 2 + lax.axis_index("c")

    pltpu.sync_copy(drv_idx_hbm.at[wid], didx_v)
    pltpu.sync_copy(tm_idx_hbm.at[wid], tidx_v)
    pltpu.sync_copy(wk_idx_hbm.at[wid], widx_v)
    pltpu.sync_copy(dist_hbm.at[wid], dist_v)
    pltpu.sync_copy(wk_tab_hbm, wk_tab_v)

    copies = []
    for j in range(NCHUNK):
        copies.append(pltpu.async_copy(
            drv_tab_hbm.at[didx_v.at[j]],
            d_rows.at[pl.ds(j * 128, 128)], sem_d))
        copies.append(pltpu.async_copy(
            tm_tab_hbm.at[tidx_v.at[j]],
            t_rows.at[pl.ds(j * 128, 128)], sem_t))
    for c in copies:
        c.wait()

    iota = lax.iota(jnp.int32, 16)

    def drv_body(i, carry):
        v = d_rows[i]
        plsc.store_scatter(out_v, [i * D_OUT + iota], v)
        return carry
    lax.fori_loop(0, BPW, drv_body, 0, unroll=8)

    def tm_body(i, carry):
        e = i * 16 + iota
        r = lax.shift_right_logical(e, 3)
        c = lax.bitwise_and(e, 7)
        v = plsc.load_gather(t_rows, [r, c])
        plsc.store_scatter(out_v, [r * D_OUT + (D_DRV + D_WEEK) + c], v)
        return carry
    lax.fori_loop(0, BPW * D_TIME // 16, tm_body, 0, unroll=8)

    def wk_body(i, carry):
        rvec = i * 16 + iota
        widx = widx_v[pl.ds(i * 16, 16)]
        for c in range(D_WEEK):
            v = plsc.load_gather(
                wk_tab_v, [widx, jnp.full((16,), c, jnp.int32)])
            plsc.store_scatter(out_v, [rvec * D_OUT + (D_DRV + c)], v)
        dv = dist_v[pl.ds(i * 16, 16)]
        dn = dv * (1.0 / DIST_STD) - (DIST_MEAN / DIST_STD)
        plsc.store_scatter(out_v, [rvec * D_OUT + (D_OUT - 1)], dn)
        return carry
    lax.fori_loop(0, BPW // 16, wk_body, 0, unroll=4)

    pltpu.sync_copy(out_v, out_hbm.at[pl.ds(wid * BPW * D_OUT, BPW * D_OUT)])


@jax.jit
def kernel(driverID, weekID, timeID, dist, driver_em, week_em, time_em):
    mesh = plsc.VectorSubcoreMesh(core_axis_name="c", subcore_axis_name="s")
    k = pl.kernel(
        _body,
        out_type=jax.ShapeDtypeStruct((B * D_OUT,), jnp.float32),
        mesh=mesh,
        scratch_types=[
            pltpu.VMEM((NCHUNK, 128), jnp.int32),
            pltpu.VMEM((NCHUNK, 128), jnp.int32),
            pltpu.VMEM((BPW,), jnp.int32),
            pltpu.VMEM((BPW,), jnp.float32),
            pltpu.VMEM((BPW, D_DRV), jnp.float32),
            pltpu.VMEM((BPW, D_TIME), jnp.float32),
            pltpu.VMEM((7, D_WEEK), jnp.float32),
            pltpu.VMEM((BPW * D_OUT,), jnp.float32),
            pltpu.SemaphoreType.DMA,
            pltpu.SemaphoreType.DMA,
        ],
    )
    out_flat = k(
        driverID.reshape(NW, NCHUNK, 128),
        weekID.reshape(NW, BPW),
        timeID.reshape(NW, NCHUNK, 128),
        dist.reshape(NW, BPW),
        driver_em,
        week_em,
        time_em,
    )
    return out_flat.reshape(B, D_OUT)


# speedup vs baseline: 3.1135x; 3.1135x over previous
import jax
import jax.numpy as jnp
from jax import lax
from jax.experimental import pallas as pl
from jax.experimental.pallas import tpu as pltpu
from jax.experimental.pallas import tpu_sc as plsc

B = 16384
D_DRV, D_WEEK, D_TIME = 16, 3, 8
D_OUT = D_DRV + D_WEEK + D_TIME + 1
NW = 32
BPW = B // NW
NCHUNK = BPW // 128

DIST_MEAN = 10.0
DIST_STD = 5.0


def _body(drv_idx_hbm, wk_idx_hbm, tm_idx_hbm, dist_hbm,
          drv_tab_hbm, wk_tab_hbm, tm_tab_hbm, out_hbm,
          didx_v, tidx_v, widx_v, dist_v, d_rows, t_rows, wk_tab_v, out_v,
          sem_d, sem_t):
    wid = lax.axis_index("s") * 2 + lax.axis_index("c")

    pltpu.sync_copy(drv_idx_hbm.at[wid], didx_v)
    pltpu.sync_copy(tm_idx_hbm.at[wid], tidx_v)
    pltpu.sync_copy(wk_idx_hbm.at[wid], widx_v)
    pltpu.sync_copy(dist_hbm.at[wid], dist_v)
    pltpu.sync_copy(wk_tab_hbm, wk_tab_v)

    copies = []
    for j in range(NCHUNK):
        copies.append(pltpu.async_copy(
            drv_tab_hbm.at[didx_v.at[j]],
            d_rows.at[pl.ds(j * 128, 128)], sem_d))
        copies.append(pltpu.async_copy(
            tm_tab_hbm.at[tidx_v.at[j]],
            t_rows.at[pl.ds(j * 128, 128)], sem_t))
    for c in copies:
        c.wait()

    iota = lax.iota(jnp.int32, 16)

    def drv_body(i, carry):
        v = d_rows[i]
        plsc.store_scatter(out_v, [i * D_OUT + iota], v)
        return carry
    lax.fori_loop(0, BPW, drv_body, 0, unroll=8)

    def tm_body(i, carry):
        e = i * 16 + iota
        r = lax.shift_right_logical(e, 3)
        c = lax.bitwise_and(e, 7)
        v = plsc.load_gather(t_rows, [r, c])
        plsc.store_scatter(out_v, [r * D_OUT + (D_DRV + D_WEEK) + c], v)
        return carry
    lax.fori_loop(0, BPW * D_TIME // 16, tm_body, 0, unroll=8)

    def wk_body(i, carry):
        rvec = i * 16 + iota
        widx = widx_v[pl.ds(i * 16, 16)]
        for c in range(D_WEEK):
            v = plsc.load_gather(
                wk_tab_v, [widx, jnp.full((16,), c, jnp.int32)])
            plsc.store_scatter(out_v, [rvec * D_OUT + (D_DRV + c)], v)
        dv = dist_v[pl.ds(i * 16, 16)]
        dn = dv * (1.0 / DIST_STD) - (DIST_MEAN / DIST_STD)
        plsc.store_scatter(out_v, [rvec * D_OUT + (D_OUT - 1)], dn)
        return carry
    lax.fori_loop(0, BPW // 16, wk_body, 0, unroll=4)

    pltpu.sync_copy(out_v, out_hbm.at[pl.ds(wid * BPW * D_OUT, BPW * D_OUT)])


@jax.jit
def kernel(driverID, weekID, timeID, dist, driver_em, week_em, time_em):
    mesh = plsc.VectorSubcoreMesh(core_axis_name="c", subcore_axis_name="s")
    k = pl.kernel(
        _body,
        out_type=jax.ShapeDtypeStruct((B * D_OUT,), jnp.float32),
        mesh=mesh,
        compiler_params=pltpu.CompilerParams(
            needs_layout_passes=False, use_tc_tiling_on_sc=False),
        scratch_types=[
            pltpu.VMEM((NCHUNK, 128), jnp.int32),
            pltpu.VMEM((NCHUNK, 128), jnp.int32),
            pltpu.VMEM((BPW,), jnp.int32),
            pltpu.VMEM((BPW,), jnp.float32),
            pltpu.VMEM((BPW, D_DRV), jnp.float32),
            pltpu.VMEM((BPW, D_TIME), jnp.float32),
            pltpu.VMEM((7, D_WEEK), jnp.float32),
            pltpu.VMEM((BPW * D_OUT,), jnp.float32),
            pltpu.SemaphoreType.DMA,
            pltpu.SemaphoreType.DMA,
        ],
    )
    out_flat = k(
        driverID.reshape(NW, NCHUNK, 128),
        weekID.reshape(NW, BPW),
        timeID.reshape(NW, NCHUNK, 128),
        dist.reshape(NW, BPW),
        driver_em,
        week_em,
        time_em,
    )
    return out_flat.reshape(B, D_OUT)


# speedup vs baseline: 3.2480x; 1.0432x over previous
import jax
import jax.numpy as jnp
from jax import lax
from jax.experimental import pallas as pl
from jax.experimental.pallas import tpu as pltpu
from jax.experimental.pallas import tpu_sc as plsc

B = 16384
D_DRV, D_WEEK, D_TIME = 16, 3, 8
D_OUT = D_DRV + D_WEEK + D_TIME + 1
NW = 32
BPW = B // NW
NCHUNK = BPW // 128

DIST_MEAN = 10.0
DIST_STD = 5.0


def _body(drv_idx_hbm, wk_idx_hbm, tm_idx_hbm, dist_hbm,
          drv_tab_hbm, wk_tab_hbm, tm_tab_hbm, out_hbm,
          didx_v, tidx_v, widx_v, dist_v, d_rows, t_rows, wk_tab_v, out_v,
          sem_i, sem_d, sem_t):
    wid = lax.axis_index("s") * 2 + lax.axis_index("c")
    base = wid * BPW

    stage = []
    for j in range(NCHUNK):
        stage.append(pltpu.async_copy(
            drv_idx_hbm.at[pl.ds(base + j * 128, 128)], didx_v.at[j], sem_i))
        stage.append(pltpu.async_copy(
            tm_idx_hbm.at[pl.ds(base + j * 128, 128)], tidx_v.at[j], sem_i))
    stage.append(pltpu.async_copy(
        wk_idx_hbm.at[pl.ds(base, BPW)], widx_v, sem_i))
    stage.append(pltpu.async_copy(
        dist_hbm.at[pl.ds(base, BPW)], dist_v, sem_i))
    stage.append(pltpu.async_copy(wk_tab_hbm, wk_tab_v, sem_i))
    for c in stage:
        c.wait()

    copies = []
    for j in range(NCHUNK):
        copies.append(pltpu.async_copy(
            drv_tab_hbm.at[didx_v.at[j]],
            d_rows.at[pl.ds(j * 128, 128)], sem_d))
        copies.append(pltpu.async_copy(
            tm_tab_hbm.at[tidx_v.at[j]],
            t_rows.at[pl.ds(j * 128, 128)], sem_t))

    iota = lax.iota(jnp.int32, 16)

    def wk_body(i, carry):
        rvec = i * 16 + iota
        widx = widx_v[pl.ds(i * 16, 16)]
        for c in range(D_WEEK):
            v = plsc.load_gather(
                wk_tab_v, [widx, jnp.full((16,), c, jnp.int32)])
            plsc.store_scatter(out_v, [rvec * D_OUT + (D_DRV + c)], v)
        dv = dist_v[pl.ds(i * 16, 16)]
        dn = dv * (1.0 / DIST_STD) - (DIST_MEAN / DIST_STD)
        plsc.store_scatter(out_v, [rvec * D_OUT + (D_OUT - 1)], dn)
        return carry
    lax.fori_loop(0, BPW // 16, wk_body, 0, unroll=4)

    for c in copies:
        c.wait()

    def drv_body(i, carry):
        v = d_rows[i]
        plsc.store_scatter(out_v, [i * D_OUT + iota], v)
        return carry
    lax.fori_loop(0, BPW, drv_body, 0, unroll=8)

    def tm_body(i, carry):
        e = i * 16 + iota
        r = lax.shift_right_logical(e, 3)
        c = lax.bitwise_and(e, 7)
        v = plsc.load_gather(t_rows, [r, c])
        plsc.store_scatter(out_v, [r * D_OUT + (D_DRV + D_WEEK) + c], v)
        return carry
    lax.fori_loop(0, BPW * D_TIME // 16, tm_body, 0, unroll=8)

    pltpu.sync_copy(out_v, out_hbm.at[pl.ds(base * D_OUT, BPW * D_OUT)])


@jax.jit
def kernel(driverID, weekID, timeID, dist, driver_em, week_em, time_em):
    mesh = plsc.VectorSubcoreMesh(core_axis_name="c", subcore_axis_name="s")
    k = pl.kernel(
        _body,
        out_type=jax.ShapeDtypeStruct((B * D_OUT,), jnp.float32),
        mesh=mesh,
        compiler_params=pltpu.CompilerParams(
            needs_layout_passes=False, use_tc_tiling_on_sc=False),
        scratch_types=[
            pltpu.VMEM((NCHUNK, 128), jnp.int32),
            pltpu.VMEM((NCHUNK, 128), jnp.int32),
            pltpu.VMEM((BPW,), jnp.int32),
            pltpu.VMEM((BPW,), jnp.float32),
            pltpu.VMEM((BPW, D_DRV), jnp.float32),
            pltpu.VMEM((BPW, D_TIME), jnp.float32),
            pltpu.VMEM((7, D_WEEK), jnp.float32),
            pltpu.VMEM((BPW * D_OUT,), jnp.float32),
            pltpu.SemaphoreType.DMA,
            pltpu.SemaphoreType.DMA,
            pltpu.SemaphoreType.DMA,
        ],
    )
    out_flat = k(driverID, weekID, timeID, dist,
                 driver_em, week_em, time_em)
    return out_flat.reshape(B, D_OUT)


# speedup vs baseline: 5.0994x; 1.5700x over previous
import jax
import jax.numpy as jnp
from jax import lax
from jax.experimental import pallas as pl
from jax.experimental.pallas import tpu as pltpu
from jax.experimental.pallas import tpu_sc as plsc

B = 16384
D_DRV, D_WEEK, D_TIME = 16, 3, 8
D_OUT = D_DRV + D_WEEK + D_TIME + 1
V_DRV, V_TIME = 24000, 1440
NITER = B // 16

DIST_MEAN = 10.0
DIST_STD = 5.0


def _body(drv_idx_hbm, wk_idx_hbm, tm_idx_hbm, dist_hbm,
          drv_tab_hbm, wk_tab_hbm, tm_tab_hbm, out_hbm,
          tab_v, wk_tab_v, idx_v, val_v, out_v, sem):
    wid = lax.axis_index("s") * 2 + lax.axis_index("c")

    def gather_loop(tab_ref):
        def body(i, carry):
            idx = idx_v[pl.ds(i * 16, 16)]
            out_v[pl.ds(i * 16, 16)] = plsc.load_gather(tab_ref, [idx])
            return carry
        lax.fori_loop(0, NITER, body, 0, unroll=8)

    @pl.when(wid < D_DRV)
    def _():
        c1 = pltpu.async_copy(drv_tab_hbm.at[wid], tab_v, sem)
        c2 = pltpu.async_copy(drv_idx_hbm, idx_v, sem)
        c1.wait()
        c2.wait()
        gather_loop(tab_v)

    @pl.when(jnp.logical_and(wid >= D_DRV, wid < D_DRV + D_WEEK))
    def _():
        c1 = pltpu.async_copy(wk_tab_hbm, wk_tab_v, sem)
        c2 = pltpu.async_copy(wk_idx_hbm, idx_v, sem)
        c1.wait()
        c2.wait()
        wrow = jnp.full((16,), wid - D_DRV, jnp.int32)

        def wk_body(i, carry):
            idx = idx_v[pl.ds(i * 16, 16)]
            out_v[pl.ds(i * 16, 16)] = plsc.load_gather(
                wk_tab_v, [wrow, idx])
            return carry
        lax.fori_loop(0, NITER, wk_body, 0, unroll=8)

    @pl.when(jnp.logical_and(wid >= D_DRV + D_WEEK, wid < D_OUT - 1))
    def _():
        c1 = pltpu.async_copy(tm_tab_hbm.at[wid - (D_DRV + D_WEEK)],
                              tab_v.at[pl.ds(0, V_TIME)], sem)
        c2 = pltpu.async_copy(tm_idx_hbm, idx_v, sem)
        c1.wait()
        c2.wait()
        gather_loop(tab_v)

    @pl.when(wid == D_OUT - 1)
    def _():
        pltpu.async_copy(dist_hbm, val_v, sem).wait()

        def dist_body(i, carry):
            dv = val_v[pl.ds(i * 16, 16)]
            out_v[pl.ds(i * 16, 16)] = (
                dv * (1.0 / DIST_STD) - (DIST_MEAN / DIST_STD))
            return carry
        lax.fori_loop(0, NITER, dist_body, 0, unroll=8)

    @pl.when(wid < D_OUT)
    def _():
        pltpu.sync_copy(out_v, out_hbm.at[wid])


@jax.jit
def kernel(driverID, weekID, timeID, dist, driver_em, week_em, time_em):
    mesh = plsc.VectorSubcoreMesh(core_axis_name="c", subcore_axis_name="s")
    k = pl.kernel(
        _body,
        out_type=jax.ShapeDtypeStruct((D_OUT, B), jnp.float32),
        mesh=mesh,
        compiler_params=pltpu.CompilerParams(
            needs_layout_passes=False, use_tc_tiling_on_sc=False),
        scratch_types=[
            pltpu.VMEM((V_DRV,), jnp.float32),
            pltpu.VMEM((D_WEEK, 7), jnp.float32),
            pltpu.VMEM((B,), jnp.int32),
            pltpu.VMEM((B,), jnp.float32),
            pltpu.VMEM((B,), jnp.float32),
            pltpu.SemaphoreType.DMA,
        ],
    )
    out_t = k(driverID, weekID, timeID, dist,
              driver_em.T, week_em.T, time_em.T)
    return out_t.T


# speedup vs baseline: 6.2738x; 1.2303x over previous
import jax
import jax.numpy as jnp
from jax import lax
from jax.experimental import pallas as pl
from jax.experimental.pallas import tpu as pltpu
from jax.experimental.pallas import tpu_sc as plsc

B = 16384
D_DRV, D_WEEK, D_TIME = 16, 3, 8
D_OUT = D_DRV + D_WEEK + D_TIME + 1
V_DRV, V_TIME = 24000, 1440
NITER = B // 16

DIST_MEAN = 10.0
DIST_STD = 5.0


def _body(drv_idx_hbm, wk_idx_hbm, tm_idx_hbm, dist_hbm,
          drv_tab_hbm, wk_tab_hbm, tm_tab_hbm, out_hbm,
          tab_v, wk_tab_v, idx_v, val_v, out_v, sem):
    wid = lax.axis_index("s") * 2 + lax.axis_index("c")

    def gather_loop(tab_ref):
        @plsc.parallel_loop(0, B, step=16, unroll=8)
        def _(i):
            idx = idx_v[pl.ds(i, 16)]
            out_v[pl.ds(i, 16)] = plsc.load_gather(tab_ref, [idx])

    @pl.when(wid < D_DRV)
    def _():
        c1 = pltpu.async_copy(drv_tab_hbm.at[wid], tab_v, sem)
        c2 = pltpu.async_copy(drv_idx_hbm, idx_v, sem)
        c1.wait()
        c2.wait()
        gather_loop(tab_v)

    @pl.when(jnp.logical_and(wid >= D_DRV, wid < D_DRV + D_WEEK))
    def _():
        c1 = pltpu.async_copy(wk_tab_hbm, wk_tab_v, sem)
        c2 = pltpu.async_copy(wk_idx_hbm, idx_v, sem)
        c1.wait()
        c2.wait()
        wrow = jnp.full((16,), wid - D_DRV, jnp.int32)

        @plsc.parallel_loop(0, B, step=16, unroll=8)
        def _(i):
            idx = idx_v[pl.ds(i, 16)]
            out_v[pl.ds(i, 16)] = plsc.load_gather(wk_tab_v, [wrow, idx])

    @pl.when(jnp.logical_and(wid >= D_DRV + D_WEEK, wid < D_OUT - 1))
    def _():
        c1 = pltpu.async_copy(tm_tab_hbm.at[wid - (D_DRV + D_WEEK)],
                              tab_v.at[pl.ds(0, V_TIME)], sem)
        c2 = pltpu.async_copy(tm_idx_hbm, idx_v, sem)
        c1.wait()
        c2.wait()
        gather_loop(tab_v)

    @pl.when(wid == D_OUT - 1)
    def _():
        pltpu.async_copy(dist_hbm, val_v, sem).wait()

        @plsc.parallel_loop(0, B, step=16, unroll=8)
        def _(i):
            dv = val_v[pl.ds(i, 16)]
            out_v[pl.ds(i, 16)] = (
                dv * (1.0 / DIST_STD) - (DIST_MEAN / DIST_STD))

    @pl.when(wid < D_OUT)
    def _():
        pltpu.sync_copy(out_v, out_hbm.at[wid])


@jax.jit
def kernel(driverID, weekID, timeID, dist, driver_em, week_em, time_em):
    mesh = plsc.VectorSubcoreMesh(core_axis_name="c", subcore_axis_name="s")
    k = pl.kernel(
        _body,
        out_type=jax.ShapeDtypeStruct((D_OUT, B), jnp.float32),
        mesh=mesh,
        compiler_params=pltpu.CompilerParams(
            needs_layout_passes=False, use_tc_tiling_on_sc=False),
        scratch_types=[
            pltpu.VMEM((V_DRV,), jnp.float32),
            pltpu.VMEM((D_WEEK, 7), jnp.float32),
            pltpu.VMEM((B,), jnp.int32),
            pltpu.VMEM((B,), jnp.float32),
            pltpu.VMEM((B,), jnp.float32),
            pltpu.SemaphoreType.DMA,
        ],
    )
    out_t = k(driverID, weekID, timeID, dist,
              driver_em.T, week_em.T, time_em.T)
    return out_t.T
